# baseline (speedup 1.0000x reference)
import jax
import jax.numpy as jnp
from jax.experimental import pallas as pl
from jax.experimental.pallas import tpu as pltpu

_N = 4096
_D_PREV = 64
_D_HID = 32
_BM = 1024
_NB = _N // _BM
_C = 4
_RC = _BM // _C


def _cell_kernel(idx_ref, x_ref, *rest):
    a_refs = rest[:_C]
    w_ref, b_ref, g_ref, be_ref, o_ref, h_s, s0_s, acc_s = rest[_C:]
    p = pl.program_id(0)
    i = pl.program_id(1)
    rows = pl.ds(i * _BM, _BM)

    def chunk_rows(j):
        return pl.ds(i * _BM + j * _RC, _RC)

    @pl.when(jnp.logical_and(p == 0, i == 0))
    def _():
        h_s[:] = (jnp.dot(x_ref[:], w_ref[:].T,
                          preferred_element_type=jnp.float32) + b_ref[:])

    @pl.when(p == 0)
    def _():
        for j in range(_C):
            s0_s[chunk_rows(j), :] = jnp.dot(
                a_refs[j][0], h_s[:], preferred_element_type=jnp.float32)

    @pl.when(p == 1)
    def _():
        f_s1 = idx_ref[3].astype(jnp.float32)
        rhs = s0_s[:] + f_s1 * h_s[:]
        for j in range(_C):
            acc_s[chunk_rows(j), :] = jnp.dot(
                a_refs[j][0], rhs, preferred_element_type=jnp.float32)

    @pl.when(jnp.logical_and(p == 2, idx_ref[3] + idx_ref[4] == 0))
    def _():
        for j in range(_C):
            acc_s[chunk_rows(j), :] += jnp.dot(
                a_refs[j][0], h_s[:], preferred_element_type=jnp.float32)

    @pl.when(p == 2)
    def _():
        f_s0 = idx_ref[4].astype(jnp.float32)
        t = acc_s[rows, :] + f_s0 * s0_s[rows, :]
        mu = jnp.mean(t, axis=-1, keepdims=True)
        var = jnp.mean((t - mu) * (t - mu), axis=-1, keepdims=True)
        ln = (t - mu) / jnp.sqrt(var + 1e-5) * g_ref[:] + be_ref[:]
        o_ref[:] = 0.5 * ln * (1.0 + jax.lax.erf(ln * (2.0 ** -0.5)))


def _adj_index_map(p, i, idx, j):
    dup = idx[3] + idx[4]
    m = jnp.where(p == 0, idx[0], jnp.where(p == 1, idx[1],
                  jnp.where(dup > 0, idx[1], idx[2])))
    row = jnp.where(jnp.logical_and(p == 2, dup > 0),
                    (_NB - 1) * _C + j, i * _C + j)
    return (m, row, 0)


def kernel(x, adjs, idxes_seq, idxes_res, connection_dict, W, b, gamma, beta):
    del connection_dict
    s0 = jnp.asarray(idxes_seq[0], jnp.int32)
    s1 = jnp.asarray(idxes_seq[1], jnp.int32)
    r0 = jnp.asarray(idxes_res[0], jnp.int32)
    f_s1 = (r0 == s1).astype(jnp.int32)
    f_s0 = jnp.logical_and(r0 == s0, r0 != s1).astype(jnp.int32)
    idx_all = jnp.stack([s0, s1, r0, f_s1, f_s0])
    adj_specs = [
        pl.BlockSpec((1, _RC, _N),
                     lambda p, i, idx, j=j: _adj_index_map(p, i, idx, j))
        for j in range(_C)
    ]
    grid_spec = pltpu.PrefetchScalarGridSpec(
        num_scalar_prefetch=1,
        grid=(3, _NB),
        in_specs=[
            pl.BlockSpec((_N, _D_PREV), lambda p, i, idx: (0, 0)),
            *adj_specs,
            pl.BlockSpec((_D_HID, _D_PREV), lambda p, i, idx: (0, 0)),
            pl.BlockSpec((1, _D_HID), lambda p, i, idx: (0, 0)),
            pl.BlockSpec((1, _D_HID), lambda p, i, idx: (0, 0)),
            pl.BlockSpec((1, _D_HID), lambda p, i, idx: (0, 0)),
        ],
        out_specs=pl.BlockSpec((_BM, _D_HID), lambda p, i, idx: (i, 0)),
        scratch_shapes=[
            pltpu.VMEM((_N, _D_HID), jnp.float32),
            pltpu.VMEM((_N, _D_HID), jnp.float32),
            pltpu.VMEM((_N, _D_HID), jnp.float32),
        ],
    )
    return pl.pallas_call(
        _cell_kernel,
        grid_spec=grid_spec,
        out_shape=jax.ShapeDtypeStruct((_N, _D_HID), jnp.float32),
    )(idx_all, x, *([adjs] * _C), W,
      b.reshape(1, _D_HID), gamma.reshape(1, _D_HID), beta.reshape(1, _D_HID))


# speedup vs baseline: 1.0149x; 1.0149x over previous
import jax
import jax.numpy as jnp
from jax.experimental import pallas as pl
from jax.experimental.pallas import tpu as pltpu

_N = 4096
_D_PREV = 64
_D_HID = 32
_BM = 1024
_NB = _N // _BM
_C = 8
_RC = _BM // _C


def _cell_kernel(idx_ref, x_ref, *rest):
    a_refs = rest[:_C]
    w_ref, b_ref, g_ref, be_ref, o_ref, h_s, s0_s, acc_s = rest[_C:]
    p = pl.program_id(0)
    i = pl.program_id(1)
    rows = pl.ds(i * _BM, _BM)

    def chunk_rows(j):
        return pl.ds(i * _BM + j * _RC, _RC)

    @pl.when(jnp.logical_and(p == 0, i == 0))
    def _():
        h_s[:] = (jnp.dot(x_ref[:], w_ref[:].T,
                          preferred_element_type=jnp.float32) + b_ref[:])

    @pl.when(p == 0)
    def _():
        for j in range(_C):
            s0_s[chunk_rows(j), :] = jnp.dot(
                a_refs[j][0], h_s[:], preferred_element_type=jnp.float32)

    @pl.when(p == 1)
    def _():
        f_s1 = idx_ref[3].astype(jnp.float32)
        rhs = s0_s[:] + f_s1 * h_s[:]
        for j in range(_C):
            acc_s[chunk_rows(j), :] = jnp.dot(
                a_refs[j][0], rhs, preferred_element_type=jnp.float32)

    @pl.when(jnp.logical_and(p == 2, idx_ref[3] + idx_ref[4] == 0))
    def _():
        for j in range(_C):
            acc_s[chunk_rows(j), :] += jnp.dot(
                a_refs[j][0], h_s[:], preferred_element_type=jnp.float32)

    @pl.when(p == 2)
    def _():
        f_s0 = idx_ref[4].astype(jnp.float32)
        t = acc_s[rows, :] + f_s0 * s0_s[rows, :]
        mu = jnp.mean(t, axis=-1, keepdims=True)
        var = jnp.mean((t - mu) * (t - mu), axis=-1, keepdims=True)
        ln = (t - mu) / jnp.sqrt(var + 1e-5) * g_ref[:] + be_ref[:]
        o_ref[:] = 0.5 * ln * (1.0 + jax.lax.erf(ln * (2.0 ** -0.5)))


def _adj_index_map(p, i, idx, j):
    dup = idx[3] + idx[4]
    m = jnp.where(p == 0, idx[0], jnp.where(p == 1, idx[1],
                  jnp.where(dup > 0, idx[1], idx[2])))
    row = jnp.where(jnp.logical_and(p == 2, dup > 0),
                    (_NB - 1) * _C + j, i * _C + j)
    return (m, row, 0)


def kernel(x, adjs, idxes_seq, idxes_res, connection_dict, W, b, gamma, beta):
    del connection_dict
    s0 = jnp.asarray(idxes_seq[0], jnp.int32)
    s1 = jnp.asarray(idxes_seq[1], jnp.int32)
    r0 = jnp.asarray(idxes_res[0], jnp.int32)
    f_s1 = (r0 == s1).astype(jnp.int32)
    f_s0 = jnp.logical_and(r0 == s0, r0 != s1).astype(jnp.int32)
    idx_all = jnp.stack([s0, s1, r0, f_s1, f_s0])
    adj_specs = [
        pl.BlockSpec((1, _RC, _N),
                     lambda p, i, idx, j=j: _adj_index_map(p, i, idx, j))
        for j in range(_C)
    ]
    grid_spec = pltpu.PrefetchScalarGridSpec(
        num_scalar_prefetch=1,
        grid=(3, _NB),
        in_specs=[
            pl.BlockSpec((_N, _D_PREV), lambda p, i, idx: (0, 0)),
            *adj_specs,
            pl.BlockSpec((_D_HID, _D_PREV), lambda p, i, idx: (0, 0)),
            pl.BlockSpec((1, _D_HID), lambda p, i, idx: (0, 0)),
            pl.BlockSpec((1, _D_HID), lambda p, i, idx: (0, 0)),
            pl.BlockSpec((1, _D_HID), lambda p, i, idx: (0, 0)),
        ],
        out_specs=pl.BlockSpec((_BM, _D_HID), lambda p, i, idx: (i, 0)),
        scratch_shapes=[
            pltpu.VMEM((_N, _D_HID), jnp.float32),
            pltpu.VMEM((_N, _D_HID), jnp.float32),
            pltpu.VMEM((_N, _D_HID), jnp.float32),
        ],
    )
    return pl.pallas_call(
        _cell_kernel,
        grid_spec=grid_spec,
        out_shape=jax.ShapeDtypeStruct((_N, _D_HID), jnp.float32),
    )(idx_all, x, *([adjs] * _C), W,
      b.reshape(1, _D_HID), gamma.reshape(1, _D_HID), beta.reshape(1, _D_HID))


# speedup vs baseline: 1.2407x; 1.2225x over previous
import jax
import jax.numpy as jnp
from jax.experimental import pallas as pl
from jax.experimental.pallas import tpu as pltpu

_N = 4096
_D_PREV = 64
_D_HID = 32
_BM = 1024
_NB = _N // _BM
_C = 8
_RC = _BM // _C


def _flags(seq, res):
    f_s1 = res[0] == seq[1]
    f_s0 = jnp.logical_and(res[0] == seq[0], jnp.logical_not(f_s1))
    return f_s1, f_s0


def _cell_kernel(seq_ref, res_ref, xt_ref, *rest):
    a_refs = rest[:_C]
    w_ref, b_ref, g_ref, be_ref, o_ref, h_s, s0_s, acc_s = rest[_C:]
    p = pl.program_id(0)
    i = pl.program_id(1)
    rows = pl.ds(i * _BM, _BM)
    f_s1, f_s0 = _flags(seq_ref, res_ref)

    def chunk_rows(j):
        return pl.ds(i * _BM + j * _RC, _RC)

    @pl.when(jnp.logical_and(p == 0, i == 0))
    def _():
        ht = jnp.dot(w_ref[:], xt_ref[:],
                     preferred_element_type=jnp.float32)
        h_s[:] = ht.T + b_ref[:]

    @pl.when(p == 0)
    def _():
        for j in range(_C):
            s0_s[chunk_rows(j), :] = jnp.dot(
                a_refs[j][0], h_s[:], preferred_element_type=jnp.float32)

    @pl.when(p == 1)
    def _():
        rhs = s0_s[:] + f_s1.astype(jnp.float32) * h_s[:]
        for j in range(_C):
            acc_s[chunk_rows(j), :] = jnp.dot(
                a_refs[j][0], rhs, preferred_element_type=jnp.float32)

    @pl.when(jnp.logical_and(p == 2,
                             jnp.logical_not(jnp.logical_or(f_s1, f_s0))))
    def _():
        for j in range(_C):
            acc_s[chunk_rows(j), :] += jnp.dot(
                a_refs[j][0], h_s[:], preferred_element_type=jnp.float32)

    @pl.when(p == 2)
    def _():
        t = acc_s[rows, :] + f_s0.astype(jnp.float32) * s0_s[rows, :]
        mu = jnp.mean(t, axis=-1, keepdims=True)
        var = jnp.mean((t - mu) * (t - mu), axis=-1, keepdims=True)
        ln = (t - mu) / jnp.sqrt(var + 1e-5) * g_ref[:] + be_ref[:]
        o_ref[:] = (0.5 * ln * (1.0 + jax.lax.erf(ln * (2.0 ** -0.5)))).T


def _adj_index_map(p, i, seq, res, j):
    f_s1, f_s0 = _flags(seq, res)
    dup = jnp.logical_or(f_s1, f_s0)
    m = jnp.where(p == 0, seq[0], jnp.where(p == 1, seq[1],
                  jnp.where(dup, seq[1], res[0])))
    row = jnp.where(jnp.logical_and(p == 2, dup),
                    (_NB - 1) * _C + j, i * _C + j)
    return (m, row, 0)


def kernel(x, adjs, idxes_seq, idxes_res, connection_dict, W, b, gamma, beta):
    del connection_dict
    seq = jnp.asarray(idxes_seq, jnp.int32)
    res = jnp.asarray(idxes_res, jnp.int32)
    adj_specs = [
        pl.BlockSpec((1, _RC, _N),
                     lambda p, i, seq, res, j=j: _adj_index_map(p, i, seq,
                                                                res, j))
        for j in range(_C)
    ]
    grid_spec = pltpu.PrefetchScalarGridSpec(
        num_scalar_prefetch=2,
        grid=(3, _NB),
        in_specs=[
            pl.BlockSpec((_D_PREV, _N), lambda p, i, seq, res: (0, 0)),
            *adj_specs,
            pl.BlockSpec((_D_HID, _D_PREV), lambda p, i, seq, res: (0, 0)),
            pl.BlockSpec((1, _D_HID), lambda p, i, seq, res: (0, 0)),
            pl.BlockSpec((1, _D_HID), lambda p, i, seq, res: (0, 0)),
            pl.BlockSpec((1, _D_HID), lambda p, i, seq, res: (0, 0)),
        ],
        out_specs=pl.BlockSpec((_D_HID, _BM), lambda p, i, seq, res: (0, i)),
        scratch_shapes=[
            pltpu.VMEM((_N, _D_HID), jnp.float32),
            pltpu.VMEM((_N, _D_HID), jnp.float32),
            pltpu.VMEM((_N, _D_HID), jnp.float32),
        ],
    )
    out_t = pl.pallas_call(
        _cell_kernel,
        grid_spec=grid_spec,
        out_shape=jax.ShapeDtypeStruct((_D_HID, _N), jnp.float32),
    )(seq, res, x.T, *([adjs] * _C), W,
      b.reshape(1, _D_HID), gamma.reshape(1, _D_HID), beta.reshape(1, _D_HID))
    return out_t.T


# speedup vs baseline: 1.2827x; 1.0339x over previous
import jax
import jax.numpy as jnp
from jax.experimental import pallas as pl
from jax.experimental.pallas import tpu as pltpu

_N = 4096
_D_PREV = 64
_D_HID = 32
_BM = 1024
_NB = _N // _BM
_C = 8
_RC = _BM // _C


def _flags(seq, res):
    f_s1 = res[0] == seq[1]
    f_s0 = jnp.logical_and(res[0] == seq[0], jnp.logical_not(f_s1))
    return f_s1, f_s0


def _cell_kernel(seq_ref, res_ref, xt_ref, *rest):
    a_refs = rest[:_C]
    w_ref, b_ref, g_ref, be_ref, o_ref, h_s, s0_s, acc_s = rest[_C:]
    p = pl.program_id(0)
    i = pl.program_id(1)
    rows = pl.ds(i * _BM, _BM)
    f_s1, f_s0 = _flags(seq_ref, res_ref)

    def chunk_rows(j):
        return pl.ds(i * _BM + j * _RC, _RC)

    @pl.when(jnp.logical_and(p == 0, i == 0))
    def _():
        ht = jnp.dot(w_ref[:], xt_ref[:],
                     preferred_element_type=jnp.float32)
        h_s[:] = ht.T + b_ref[:]

    @pl.when(p == 0)
    def _():
        for j in range(_C):
            s0_s[chunk_rows(j), :] = jnp.dot(
                a_refs[j][0], h_s[:], preferred_element_type=jnp.float32)

    def epilogue(t):
        mu = jnp.mean(t, axis=-1, keepdims=True)
        var = jnp.mean((t - mu) * (t - mu), axis=-1, keepdims=True)
        ln = (t - mu) / jnp.sqrt(var + 1e-5) * g_ref[:] + be_ref[:]
        return 0.5 * ln * (1.0 + jax.lax.erf(ln * (2.0 ** -0.5)))

    dup = jnp.logical_or(f_s1, f_s0)

    @pl.when(p == 1)
    def _():
        rhs = s0_s[:] + f_s1.astype(jnp.float32) * h_s[:]
        for j in range(_C):
            t = jnp.dot(a_refs[j][0], rhs, preferred_element_type=jnp.float32)
            t = jnp.where(dup,
                          epilogue(t + f_s0.astype(jnp.float32)
                                   * s0_s[chunk_rows(j), :]),
                          t)
            acc_s[chunk_rows(j), :] = t

    @pl.when(jnp.logical_and(p == 2, jnp.logical_not(dup)))
    def _():
        for j in range(_C):
            acc_s[chunk_rows(j), :] += jnp.dot(
                a_refs[j][0], h_s[:], preferred_element_type=jnp.float32)
        o_ref[:] = epilogue(acc_s[rows, :]).T

    @pl.when(jnp.logical_and(p == 2, dup))
    def _():
        o_ref[:] = acc_s[rows, :].T


def _adj_index_map(p, i, seq, res, j):
    f_s1, f_s0 = _flags(seq, res)
    dup = jnp.logical_or(f_s1, f_s0)
    m = jnp.where(p == 0, seq[0], jnp.where(p == 1, seq[1],
                  jnp.where(dup, seq[1], res[0])))
    row = jnp.where(jnp.logical_and(p == 2, dup),
                    (_NB - 1) * _C + j, i * _C + j)
    return (m, row, 0)


def kernel(x, adjs, idxes_seq, idxes_res, connection_dict, W, b, gamma, beta):
    del connection_dict
    seq = jnp.asarray(idxes_seq, jnp.int32)
    res = jnp.asarray(idxes_res, jnp.int32)
    adj_specs = [
        pl.BlockSpec((1, _RC, _N),
                     lambda p, i, seq, res, j=j: _adj_index_map(p, i, seq,
                                                                res, j))
        for j in range(_C)
    ]
    grid_spec = pltpu.PrefetchScalarGridSpec(
        num_scalar_prefetch=2,
        grid=(3, _NB),
        in_specs=[
            pl.BlockSpec((_D_PREV, _N), lambda p, i, seq, res: (0, 0)),
            *adj_specs,
            pl.BlockSpec((_D_HID, _D_PREV), lambda p, i, seq, res: (0, 0)),
            pl.BlockSpec((1, _D_HID), lambda p, i, seq, res: (0, 0)),
            pl.BlockSpec((1, _D_HID), lambda p, i, seq, res: (0, 0)),
            pl.BlockSpec((1, _D_HID), lambda p, i, seq, res: (0, 0)),
        ],
        out_specs=pl.BlockSpec((_D_HID, _BM), lambda p, i, seq, res: (0, i)),
        scratch_shapes=[
            pltpu.VMEM((_N, _D_HID), jnp.float32),
            pltpu.VMEM((_N, _D_HID), jnp.float32),
            pltpu.VMEM((_N, _D_HID), jnp.float32),
        ],
    )
    out_t = pl.pallas_call(
        _cell_kernel,
        grid_spec=grid_spec,
        out_shape=jax.ShapeDtypeStruct((_D_HID, _N), jnp.float32),
    )(seq, res, x.T, *([adjs] * _C), W,
      b.reshape(1, _D_HID), gamma.reshape(1, _D_HID), beta.reshape(1, _D_HID))
    return out_t.T
